# baseline (speedup 1.0000x reference)
import functools

import jax
import jax.numpy as jnp
from jax import lax
from jax.experimental import pallas as pl
from jax.experimental.pallas import tpu as pltpu
from jax.experimental.pallas import tpu_sc as plsc

N = 10000
DIN = 128
H = 64
W0 = 80
OUT = 2
E = 320000
CHUNK = 125
NC = 2
NS = 16
NW = NC * NS
C = 80
RPT = N // NS
ZCH = 125

_F32 = jnp.float32


def _sc_aggregate(p, src_t, dst_t, zeros_w, aw):
  mesh = plsc.VectorSubcoreMesh(core_axis_name="c", subcore_axis_name="s")

  @functools.partial(
      pl.kernel,
      out_type=jax.ShapeDtypeStruct((NC, N, aw), _F32),
      mesh=mesh,
      scratch_types=[
          pltpu.VMEM((C, CHUNK), jnp.int32),
          pltpu.VMEM((C, CHUNK), jnp.int32),
          pltpu.VMEM((CHUNK, aw), _F32),
          pltpu.VMEM((CHUNK, aw), _F32),
          pltpu.VMEM((ZCH, aw), _F32),
          pltpu.VMEM_SHARED((N, aw), _F32),
          pltpu.SemaphoreType.DMA,
          pltpu.SemaphoreType.DMA,
          pltpu.SemaphoreType.DMA,
          pltpu.SemaphoreType.DMA,
      ],
      compiler_params=pltpu.CompilerParams(use_tc_tiling_on_sc=False),
      name="sc_edge_aggregate%d" % aw)
  def run(p_hbm, src_hbm, dst_hbm, z_hbm, acc_out,
          src_v, dst_v, buf0, buf1, zstage, acc_sh, sem0, sem1, ssem0, ssem1):
    c = lax.axis_index("c")
    s = lax.axis_index("s")
    w = c * NS + s
    r0 = s * RPT

    def zbody(k, carry):
      rk = r0 + k * ZCH
      pltpu.sync_copy(z_hbm.at[pl.ds(rk, ZCH)], zstage)
      pltpu.sync_copy(zstage, acc_sh.at[pl.ds(rk, ZCH)])
      return carry

    lax.fori_loop(0, RPT // ZCH, zbody, 0)
    pltpu.sync_copy(src_hbm.at[w], src_v)
    pltpu.sync_copy(dst_hbm.at[w], dst_v)
    plsc.subcore_barrier()

    pltpu.async_copy(p_hbm.at[src_v.at[0]], buf0, sem0)
    pltpu.async_copy(p_hbm.at[src_v.at[1]], buf1, sem1)

    def body(kk, carry):
      j0 = 2 * kk
      pltpu.make_async_copy(p_hbm.at[src_v.at[j0]], buf0, sem0).wait()
      pltpu.async_copy(buf0, acc_sh.at[dst_v.at[j0]], ssem0, add=True)
      pltpu.make_async_copy(p_hbm.at[src_v.at[j0 + 1]], buf1, sem1).wait()
      pltpu.async_copy(buf1, acc_sh.at[dst_v.at[j0 + 1]], ssem1, add=True)

      @pl.when(kk + 1 < C // 2)
      def _():
        pltpu.make_async_copy(buf0, acc_sh.at[dst_v.at[j0]], ssem0).wait()
        pltpu.async_copy(p_hbm.at[src_v.at[j0 + 2]], buf0, sem0)
        pltpu.make_async_copy(buf1, acc_sh.at[dst_v.at[j0 + 1]], ssem1).wait()
        pltpu.async_copy(p_hbm.at[src_v.at[j0 + 3]], buf1, sem1)

      return carry

    lax.fori_loop(0, C // 2, body, 0)
    pltpu.make_async_copy(buf0, acc_sh.at[dst_v.at[C - 2]], ssem0).wait()
    pltpu.make_async_copy(buf1, acc_sh.at[dst_v.at[C - 1]], ssem1).wait()
    plsc.subcore_barrier()

    def rbody(k, carry):
      rk = r0 + k * ZCH
      pltpu.sync_copy(acc_sh.at[pl.ds(rk, ZCH)], zstage)
      pltpu.sync_copy(zstage, acc_out.at[c, pl.ds(rk, ZCH)])
      return carry

    lax.fori_loop(0, RPT // ZCH, rbody, 0)

  return run(p, src_t, dst_t, zeros_w)


def _dot(a, b):
  return jnp.dot(a, b, preferred_element_type=_F32)


def _tc_proj(x, wlT, wrT, bl):
  def body(x_ref, wl_ref, wr_ref, bl_ref, p_ref, r_ref):
    xv = x_ref[...]
    pw = _dot(xv, wl_ref[...])
    p_ref[...] = jnp.concatenate([pw, jnp.ones((N, W0 - H), _F32)], axis=1)
    r_ref[...] = _dot(xv, wr_ref[...]) + bl_ref[...]

  return pl.pallas_call(
      body,
      out_shape=[jax.ShapeDtypeStruct((N, W0), _F32),
                 jax.ShapeDtypeStruct((N, H), _F32)],
      name="tc_proj",
  )(x, wlT, wrT, bl)


def _tc_combine(acc, inv, r, wlT, wrT, bl):
  first = inv is None

  def body(*refs):
    if first:
      acc_ref, r_ref, wl_ref, wr_ref, bl_ref, p_ref, rn_ref, inv_ref = refs
      deg = acc_ref[0, :, H:H + 1] + acc_ref[1, :, H:H + 1]
      invv = 1.0 / jnp.maximum(deg, 1.0)
      inv_ref[...] = jnp.broadcast_to(invv, (N, H))
    else:
      acc_ref, inv_in, r_ref, wl_ref, wr_ref, bl_ref, p_ref, rn_ref = refs
      invv = inv_in[...]
    a = acc_ref[0, :, :H] + acc_ref[1, :, :H]
    h = jnp.maximum(a * invv + r_ref[...], 0.0)
    p_ref[...] = _dot(h, wl_ref[...])
    rn_ref[...] = _dot(h, wr_ref[...]) + bl_ref[...]

  out_shape = [jax.ShapeDtypeStruct((N, H), _F32),
               jax.ShapeDtypeStruct((N, H), _F32)]
  if first:
    out_shape.append(jax.ShapeDtypeStruct((N, H), _F32))
    return pl.pallas_call(body, out_shape=out_shape, name="tc_combine0")(
        acc, r, wlT, wrT, bl)
  return pl.pallas_call(body, out_shape=out_shape, name="tc_combine")(
      acc, inv, r, wlT, wrT, bl)


def _tc_head(acc, inv, r, whT, bh):
  def body(acc_ref, inv_ref, r_ref, wh_ref, bh_ref, o_ref):
    a = acc_ref[0] + acc_ref[1]
    h = jnp.maximum(a * inv_ref[...] + r_ref[...], 0.0)
    o_ref[...] = _dot(h, wh_ref[...]) + bh_ref[...]

  return pl.pallas_call(
      body,
      out_shape=jax.ShapeDtypeStruct((N, OUT), _F32),
      name="tc_head",
  )(acc, inv, r, whT, bh)


def kernel(x, edge_index, Wl0, bl0, Wr0, Wl1, bl1, Wr1, Wl2, bl2, Wr2,
           Wh, bh):
  src_t = edge_index[0].reshape(NW, C, CHUNK)
  dst_t = edge_index[1].reshape(NW, C, CHUNK)

  zeros_w0 = jnp.zeros((N, W0), _F32)
  zeros_h = jnp.zeros((N, H), _F32)

  wl0T, wr0T = Wl0.T, Wr0.T
  wl1T, wr1T = Wl1.T, Wr1.T
  wl2T, wr2T = Wl2.T, Wr2.T
  whT = Wh.T
  bl0r, bl1r, bl2r, bhr = (b.reshape(1, -1) for b in (bl0, bl1, bl2, bh))

  p0, r0 = _tc_proj(x, wl0T, wr0T, bl0r)
  acc0 = _sc_aggregate(p0, src_t, dst_t, zeros_w0, W0)
  p1, r1, inv = _tc_combine(acc0, None, r0, wl1T, wr1T, bl1r)
  acc1 = _sc_aggregate(p1, src_t, dst_t, zeros_h, H)
  p2, r2 = _tc_combine(acc1, inv, r1, wl2T, wr2T, bl2r)
  acc2 = _sc_aggregate(p2, src_t, dst_t, zeros_h, H)
  return _tc_head(acc2, inv, r2, whT, bhr)


# speedup vs baseline: 1.0613x; 1.0613x over previous
import functools

import jax
import jax.numpy as jnp
from jax import lax
from jax.experimental import pallas as pl
from jax.experimental.pallas import tpu as pltpu
from jax.experimental.pallas import tpu_sc as plsc

N = 10000
DIN = 128
H = 64
W0 = 80
OUT = 2
E = 320000
CHUNK = 125
NC = 2
NS = 16
NW = NC * NS
C = 80
RPT = N // NS
ZCH = 125

_F32 = jnp.float32


def _sc_aggregate(p, src_t, dst_t, zeros_w, aw):
  mesh = plsc.VectorSubcoreMesh(core_axis_name="c", subcore_axis_name="s")
  stage_table = aw == H

  scratch = [
      pltpu.VMEM((C, CHUNK), jnp.int32),
      pltpu.VMEM((C, CHUNK), jnp.int32),
      pltpu.VMEM((CHUNK, aw), _F32),
      pltpu.VMEM((CHUNK, aw), _F32),
      pltpu.VMEM((ZCH, aw), _F32),
      pltpu.VMEM_SHARED((N, aw), _F32),
      pltpu.SemaphoreType.DMA,
      pltpu.SemaphoreType.DMA,
  ]
  if stage_table:
    scratch.append(pltpu.VMEM_SHARED((N, aw), _F32))

  @functools.partial(
      pl.kernel,
      out_type=jax.ShapeDtypeStruct((NC, N, aw), _F32),
      mesh=mesh,
      scratch_types=scratch,
      compiler_params=pltpu.CompilerParams(use_tc_tiling_on_sc=False),
      name="sc_edge_aggregate%d" % aw)
  def run(*refs):
    if stage_table:
      (p_hbm, src_hbm, dst_hbm, z_hbm, acc_out,
       src_v, dst_v, buf0, buf1, zstage, acc_sh, sem0, sem1, p_sh) = refs
    else:
      (p_hbm, src_hbm, dst_hbm, z_hbm, acc_out,
       src_v, dst_v, buf0, buf1, zstage, acc_sh, sem0, sem1) = refs
      p_sh = None
    c = lax.axis_index("c")
    s = lax.axis_index("s")
    w = c * NS + s
    r0 = s * RPT

    def zbody(k, carry):
      rk = r0 + k * ZCH
      pltpu.sync_copy(z_hbm.at[pl.ds(rk, ZCH)], zstage)
      pltpu.sync_copy(zstage, acc_sh.at[pl.ds(rk, ZCH)])
      return carry

    lax.fori_loop(0, RPT // ZCH, zbody, 0)
    pltpu.sync_copy(src_hbm.at[w], src_v)
    pltpu.sync_copy(dst_hbm.at[w], dst_v)
    plsc.subcore_barrier()

    if stage_table:
      def sbody(k, carry):
        rk = r0 + k * ZCH
        pltpu.sync_copy(p_hbm.at[pl.ds(rk, ZCH)], zstage)
        pltpu.sync_copy(zstage, p_sh.at[pl.ds(rk, ZCH)])
        return carry

      lax.fori_loop(0, RPT // ZCH, sbody, 0)
      plsc.subcore_barrier()
      table = p_sh
    else:
      table = p_hbm

    pltpu.async_copy(table.at[src_v.at[0]], buf0, sem0)

    def body(kk, carry):
      j0 = 2 * kk
      pltpu.async_copy(table.at[src_v.at[j0 + 1]], buf1, sem1)
      pltpu.make_async_copy(table.at[src_v.at[j0]], buf0, sem0).wait()
      pltpu.sync_copy(buf0, acc_sh.at[dst_v.at[j0]], add=True)

      @pl.when(kk + 1 < C // 2)
      def _():
        pltpu.async_copy(table.at[src_v.at[j0 + 2]], buf0, sem0)

      pltpu.make_async_copy(table.at[src_v.at[j0 + 1]], buf1, sem1).wait()
      pltpu.sync_copy(buf1, acc_sh.at[dst_v.at[j0 + 1]], add=True)
      return carry

    lax.fori_loop(0, C // 2, body, 0)
    plsc.subcore_barrier()

    def rbody(k, carry):
      rk = r0 + k * ZCH
      pltpu.sync_copy(acc_sh.at[pl.ds(rk, ZCH)], zstage)
      pltpu.sync_copy(zstage, acc_out.at[c, pl.ds(rk, ZCH)])
      return carry

    lax.fori_loop(0, RPT // ZCH, rbody, 0)

  return run(p, src_t, dst_t, zeros_w)


def _dot(a, b):
  return jnp.dot(a, b, preferred_element_type=_F32)


def _tc_proj(x, wlT, wrT, bl):
  def body(x_ref, wl_ref, wr_ref, bl_ref, p_ref, r_ref):
    xv = x_ref[...]
    pw = _dot(xv, wl_ref[...])
    p_ref[...] = jnp.concatenate([pw, jnp.ones((N, W0 - H), _F32)], axis=1)
    r_ref[...] = _dot(xv, wr_ref[...]) + bl_ref[...]

  return pl.pallas_call(
      body,
      out_shape=[jax.ShapeDtypeStruct((N, W0), _F32),
                 jax.ShapeDtypeStruct((N, H), _F32)],
      name="tc_proj",
  )(x, wlT, wrT, bl)


def _tc_combine(acc, inv, r, wlT, wrT, bl):
  first = inv is None

  def body(*refs):
    if first:
      acc_ref, r_ref, wl_ref, wr_ref, bl_ref, p_ref, rn_ref, inv_ref = refs
      deg = acc_ref[0, :, H:H + 1] + acc_ref[1, :, H:H + 1]
      invv = 1.0 / jnp.maximum(deg, 1.0)
      inv_ref[...] = jnp.broadcast_to(invv, (N, H))
    else:
      acc_ref, inv_in, r_ref, wl_ref, wr_ref, bl_ref, p_ref, rn_ref = refs
      invv = inv_in[...]
    a = acc_ref[0, :, :H] + acc_ref[1, :, :H]
    h = jnp.maximum(a * invv + r_ref[...], 0.0)
    p_ref[...] = _dot(h, wl_ref[...])
    rn_ref[...] = _dot(h, wr_ref[...]) + bl_ref[...]

  out_shape = [jax.ShapeDtypeStruct((N, H), _F32),
               jax.ShapeDtypeStruct((N, H), _F32)]
  if first:
    out_shape.append(jax.ShapeDtypeStruct((N, H), _F32))
    return pl.pallas_call(body, out_shape=out_shape, name="tc_combine0")(
        acc, r, wlT, wrT, bl)
  return pl.pallas_call(body, out_shape=out_shape, name="tc_combine")(
      acc, inv, r, wlT, wrT, bl)


def _tc_head(acc, inv, r, whT, bh):
  def body(acc_ref, inv_ref, r_ref, wh_ref, bh_ref, o_ref):
    a = acc_ref[0] + acc_ref[1]
    h = jnp.maximum(a * inv_ref[...] + r_ref[...], 0.0)
    o_ref[...] = _dot(h, wh_ref[...]) + bh_ref[...]

  return pl.pallas_call(
      body,
      out_shape=jax.ShapeDtypeStruct((N, OUT), _F32),
      name="tc_head",
  )(acc, inv, r, whT, bh)


def kernel(x, edge_index, Wl0, bl0, Wr0, Wl1, bl1, Wr1, Wl2, bl2, Wr2,
           Wh, bh):
  src_t = edge_index[0].reshape(NW, C, CHUNK)
  dst_t = edge_index[1].reshape(NW, C, CHUNK)

  zeros_w0 = jnp.zeros((N, W0), _F32)
  zeros_h = jnp.zeros((N, H), _F32)

  wl0T, wr0T = Wl0.T, Wr0.T
  wl1T, wr1T = Wl1.T, Wr1.T
  wl2T, wr2T = Wl2.T, Wr2.T
  whT = Wh.T
  bl0r, bl1r, bl2r, bhr = (b.reshape(1, -1) for b in (bl0, bl1, bl2, bh))

  p0, r0 = _tc_proj(x, wl0T, wr0T, bl0r)
  acc0 = _sc_aggregate(p0, src_t, dst_t, zeros_w0, W0)
  p1, r1, inv = _tc_combine(acc0, None, r0, wl1T, wr1T, bl1r)
  acc1 = _sc_aggregate(p1, src_t, dst_t, zeros_h, H)
  p2, r2 = _tc_combine(acc1, inv, r1, wl2T, wr2T, bl2r)
  acc2 = _sc_aggregate(p2, src_t, dst_t, zeros_h, H)
  return _tc_head(acc2, inv, r2, whT, bhr)


# speedup vs baseline: 1.1189x; 1.0542x over previous
import functools

import jax
import jax.numpy as jnp
from jax import lax
from jax.experimental import pallas as pl
from jax.experimental.pallas import tpu as pltpu
from jax.experimental.pallas import tpu_sc as plsc

N = 10000
DIN = 128
H = 64
W0 = 80
OUT = 2
E = 320000
CHUNK = 125
NC = 2
NS = 16
NW = NC * NS
C = 80
RPT = N // NS
ZCH = 125

_F32 = jnp.float32


def _sc_aggregate(p, src_t, dst_t, zeros_w, aw):
  mesh = plsc.VectorSubcoreMesh(core_axis_name="c", subcore_axis_name="s")

  @functools.partial(
      pl.kernel,
      out_type=jax.ShapeDtypeStruct((NC, N, aw), _F32),
      mesh=mesh,
      scratch_types=[
          pltpu.VMEM((C, CHUNK), jnp.int32),
          pltpu.VMEM((C, CHUNK), jnp.int32),
          pltpu.VMEM((CHUNK, aw), _F32),
          pltpu.VMEM((CHUNK, aw), _F32),
          pltpu.VMEM((ZCH, aw), _F32),
          pltpu.VMEM_SHARED((N, aw), _F32),
          pltpu.SemaphoreType.DMA,
          pltpu.SemaphoreType.DMA,
      ],
      compiler_params=pltpu.CompilerParams(use_tc_tiling_on_sc=False),
      name="sc_edge_aggregate%d" % aw)
  def run(p_hbm, src_hbm, dst_hbm, z_hbm, acc_out,
          src_v, dst_v, buf0, buf1, zstage, acc_sh, sem0, sem1):
    c = lax.axis_index("c")
    s = lax.axis_index("s")
    w = c * NS + s
    r0 = s * RPT

    def zbody(k, carry):
      rk = r0 + k * ZCH
      pltpu.sync_copy(z_hbm.at[pl.ds(rk, ZCH)], zstage)
      pltpu.sync_copy(zstage, acc_sh.at[pl.ds(rk, ZCH)])
      return carry

    lax.fori_loop(0, RPT // ZCH, zbody, 0)
    pltpu.sync_copy(src_hbm.at[w], src_v)
    pltpu.sync_copy(dst_hbm.at[w], dst_v)
    plsc.subcore_barrier()

    pltpu.async_copy(p_hbm.at[src_v.at[0]], buf0, sem0)

    def body(kk, carry):
      j0 = 2 * kk
      pltpu.async_copy(p_hbm.at[src_v.at[j0 + 1]], buf1, sem1)
      pltpu.make_async_copy(p_hbm.at[src_v.at[j0]], buf0, sem0).wait()
      pltpu.sync_copy(buf0, acc_sh.at[dst_v.at[j0]], add=True)

      @pl.when(kk + 1 < C // 2)
      def _():
        pltpu.async_copy(p_hbm.at[src_v.at[j0 + 2]], buf0, sem0)

      pltpu.make_async_copy(p_hbm.at[src_v.at[j0 + 1]], buf1, sem1).wait()
      pltpu.sync_copy(buf1, acc_sh.at[dst_v.at[j0 + 1]], add=True)
      return carry

    lax.fori_loop(0, C // 2, body, 0)
    plsc.subcore_barrier()

    def rbody(k, carry):
      rk = r0 + k * ZCH
      pltpu.sync_copy(acc_sh.at[pl.ds(rk, ZCH)], zstage)
      pltpu.sync_copy(zstage, acc_out.at[c, pl.ds(rk, ZCH)])
      return carry

    lax.fori_loop(0, RPT // ZCH, rbody, 0)

  return run(p, src_t, dst_t, zeros_w)


def _dot(a, b):
  return jnp.dot(a, b, preferred_element_type=_F32)


B = 1000
_PAR = pltpu.CompilerParams(
    dimension_semantics=(pltpu.PARALLEL,))


def _row_spec(w):
  return pl.BlockSpec((B, w), lambda i: (i, 0))


def _full_spec(a, b):
  return pl.BlockSpec((a, b), lambda i: (0, 0))


def _acc_spec(w):
  return pl.BlockSpec((NC, B, w), lambda i: (0, i, 0))


def _tc_proj(x, wlT, wrT, bl):
  def body(x_ref, wl_ref, wr_ref, bl_ref, p_ref, r_ref):
    xv = x_ref[...]
    pw = _dot(xv, wl_ref[...])
    p_ref[...] = jnp.concatenate([pw, jnp.ones((B, W0 - H), _F32)], axis=1)
    r_ref[...] = _dot(xv, wr_ref[...]) + bl_ref[...]

  return pl.pallas_call(
      body,
      grid=(N // B,),
      in_specs=[_row_spec(DIN), _full_spec(DIN, H), _full_spec(DIN, H),
                _full_spec(1, H)],
      out_specs=[_row_spec(W0), _row_spec(H)],
      out_shape=[jax.ShapeDtypeStruct((N, W0), _F32),
                 jax.ShapeDtypeStruct((N, H), _F32)],
      compiler_params=_PAR,
      name="tc_proj",
  )(x, wlT, wrT, bl)


def _tc_combine(acc, inv, r, wlT, wrT, bl):
  first = inv is None
  aw = W0 if first else H

  def body(*refs):
    if first:
      acc_ref, r_ref, wl_ref, wr_ref, bl_ref, p_ref, rn_ref, inv_ref = refs
      deg = acc_ref[0, :, H:H + 1] + acc_ref[1, :, H:H + 1]
      invv = 1.0 / jnp.maximum(deg, 1.0)
      inv_ref[...] = jnp.broadcast_to(invv, (B, H))
    else:
      acc_ref, inv_in, r_ref, wl_ref, wr_ref, bl_ref, p_ref, rn_ref = refs
      invv = inv_in[...]
    a = acc_ref[0, :, :H] + acc_ref[1, :, :H]
    h = jnp.maximum(a * invv + r_ref[...], 0.0)
    p_ref[...] = _dot(h, wl_ref[...])
    rn_ref[...] = _dot(h, wr_ref[...]) + bl_ref[...]

  out_shape = [jax.ShapeDtypeStruct((N, H), _F32),
               jax.ShapeDtypeStruct((N, H), _F32)]
  out_specs = [_row_spec(H), _row_spec(H)]
  in_specs = [_acc_spec(aw)]
  if not first:
    in_specs.append(_row_spec(H))
  in_specs += [_row_spec(H), _full_spec(H, H), _full_spec(H, H),
               _full_spec(1, H)]
  if first:
    out_shape.append(jax.ShapeDtypeStruct((N, H), _F32))
    out_specs.append(_row_spec(H))
    return pl.pallas_call(
        body, grid=(N // B,), in_specs=in_specs, out_specs=out_specs,
        out_shape=out_shape, compiler_params=_PAR, name="tc_combine0")(
            acc, r, wlT, wrT, bl)
  return pl.pallas_call(
      body, grid=(N // B,), in_specs=in_specs, out_specs=out_specs,
      out_shape=out_shape, compiler_params=_PAR, name="tc_combine")(
          acc, inv, r, wlT, wrT, bl)


def _tc_head(acc, inv, r, whT, bh):
  def body(acc_ref, inv_ref, r_ref, wh_ref, bh_ref, o_ref):
    a = acc_ref[0] + acc_ref[1]
    h = jnp.maximum(a * inv_ref[...] + r_ref[...], 0.0)
    o_ref[...] = _dot(h, wh_ref[...]) + bh_ref[...]

  return pl.pallas_call(
      body,
      grid=(N // B,),
      in_specs=[_acc_spec(H), _row_spec(H), _row_spec(H),
                _full_spec(H, OUT), _full_spec(1, OUT)],
      out_specs=_row_spec(OUT),
      out_shape=jax.ShapeDtypeStruct((N, OUT), _F32),
      compiler_params=_PAR,
      name="tc_head",
  )(acc, inv, r, whT, bh)


def kernel(x, edge_index, Wl0, bl0, Wr0, Wl1, bl1, Wr1, Wl2, bl2, Wr2,
           Wh, bh):
  src_t = edge_index[0].reshape(NW, C, CHUNK)
  dst_t = edge_index[1].reshape(NW, C, CHUNK)

  zeros_w0 = jnp.zeros((N, W0), _F32)
  zeros_h = jnp.zeros((N, H), _F32)

  wl0T, wr0T = Wl0.T, Wr0.T
  wl1T, wr1T = Wl1.T, Wr1.T
  wl2T, wr2T = Wl2.T, Wr2.T
  whT = Wh.T
  bl0r, bl1r, bl2r, bhr = (b.reshape(1, -1) for b in (bl0, bl1, bl2, bh))

  p0, r0 = _tc_proj(x, wl0T, wr0T, bl0r)
  acc0 = _sc_aggregate(p0, src_t, dst_t, zeros_w0, W0)
  p1, r1, inv = _tc_combine(acc0, None, r0, wl1T, wr1T, bl1r)
  acc1 = _sc_aggregate(p1, src_t, dst_t, zeros_h, H)
  p2, r2 = _tc_combine(acc1, inv, r1, wl2T, wr2T, bl2r)
  acc2 = _sc_aggregate(p2, src_t, dst_t, zeros_h, H)
  return _tc_head(acc2, inv, r2, whT, bhr)


# speedup vs baseline: 1.3358x; 1.1939x over previous
import functools

import jax
import jax.numpy as jnp
from jax import lax
from jax.experimental import pallas as pl
from jax.experimental.pallas import tpu as pltpu
from jax.experimental.pallas import tpu_sc as plsc

N = 10000
DIN = 128
H = 64
W0 = 80
OUT = 2
E = 320000
CHUNK = 100
NC = 2
NS = 16
NW = NC * NS
C = 100
RPT = N // NS
ZCH = 125

_F32 = jnp.float32


def _sc_aggregate(p, src_t, dst_t, zeros_w, aw):
  mesh = plsc.VectorSubcoreMesh(core_axis_name="c", subcore_axis_name="s")

  @functools.partial(
      pl.kernel,
      out_type=jax.ShapeDtypeStruct((NC, N, aw), _F32),
      mesh=mesh,
      scratch_types=[
          pltpu.VMEM((C, CHUNK), jnp.int32),
          pltpu.VMEM((C, CHUNK), jnp.int32),
          pltpu.VMEM((CHUNK, aw), _F32),
          pltpu.VMEM((CHUNK, aw), _F32),
          pltpu.VMEM((CHUNK, aw), _F32),
          pltpu.VMEM((CHUNK, aw), _F32),
          pltpu.VMEM((ZCH, aw), _F32),
          pltpu.VMEM_SHARED((N, aw), _F32),
          pltpu.SemaphoreType.DMA,
          pltpu.SemaphoreType.DMA,
          pltpu.SemaphoreType.DMA,
          pltpu.SemaphoreType.DMA,
      ],
      compiler_params=pltpu.CompilerParams(use_tc_tiling_on_sc=False),
      name="sc_edge_aggregate%d" % aw)
  def run(p_hbm, src_hbm, dst_hbm, z_hbm, acc_out,
          src_v, dst_v, buf0, buf1, buf2, buf3, zstage, acc_sh,
          sem0, sem1, sem2, sem3):
    c = lax.axis_index("c")
    s = lax.axis_index("s")
    w = c * NS + s
    r0 = s * RPT

    def zbody(k, carry):
      rk = r0 + k * ZCH
      pltpu.sync_copy(z_hbm.at[pl.ds(rk, ZCH)], zstage)
      pltpu.sync_copy(zstage, acc_sh.at[pl.ds(rk, ZCH)])
      return carry

    lax.fori_loop(0, RPT // ZCH, zbody, 0)
    pltpu.sync_copy(src_hbm.at[w], src_v)
    pltpu.sync_copy(dst_hbm.at[w], dst_v)
    plsc.subcore_barrier()

    bufs = (buf0, buf1, buf2, buf3)
    sems = (sem0, sem1, sem2, sem3)
    for b in range(4):
      pltpu.async_copy(p_hbm.at[src_v.at[b]], bufs[b], sems[b])

    def body(kk, carry):
      j0 = 4 * kk
      for b in range(4):
        j = j0 + b
        pltpu.make_async_copy(p_hbm.at[src_v.at[j]], bufs[b], sems[b]).wait()
        pltpu.sync_copy(bufs[b], acc_sh.at[dst_v.at[j]], add=True)

        @pl.when(j + 4 < C)
        def _():
          pltpu.async_copy(p_hbm.at[src_v.at[j + 4]], bufs[b], sems[b])

      return carry

    lax.fori_loop(0, C // 4, body, 0)
    plsc.subcore_barrier()

    def rbody(k, carry):
      rk = r0 + k * ZCH
      pltpu.sync_copy(acc_sh.at[pl.ds(rk, ZCH)], zstage)
      pltpu.sync_copy(zstage, acc_out.at[c, pl.ds(rk, ZCH)])
      return carry

    lax.fori_loop(0, RPT // ZCH, rbody, 0)

  return run(p, src_t, dst_t, zeros_w)


def _dot(a, b):
  return jnp.dot(a, b, preferred_element_type=_F32)


def _tc_proj(x, wlT, wrT, bl):
  def body(x_ref, wl_ref, wr_ref, bl_ref, p_ref, r_ref):
    xv = x_ref[...]
    pw = _dot(xv, wl_ref[...])
    p_ref[...] = jnp.concatenate([pw, jnp.ones((N, W0 - H), _F32)], axis=1)
    r_ref[...] = _dot(xv, wr_ref[...]) + bl_ref[...]

  return pl.pallas_call(
      body,
      out_shape=[jax.ShapeDtypeStruct((N, W0), _F32),
                 jax.ShapeDtypeStruct((N, H), _F32)],
      name="tc_proj",
  )(x, wlT, wrT, bl)


def _tc_combine(acc, inv, r, wlT, wrT, bl):
  first = inv is None

  def body(*refs):
    if first:
      acc_ref, r_ref, wl_ref, wr_ref, bl_ref, p_ref, rn_ref, inv_ref = refs
      deg = acc_ref[0, :, H:H + 1] + acc_ref[1, :, H:H + 1]
      invv = 1.0 / jnp.maximum(deg, 1.0)
      inv_ref[...] = jnp.broadcast_to(invv, (N, H))
    else:
      acc_ref, inv_in, r_ref, wl_ref, wr_ref, bl_ref, p_ref, rn_ref = refs
      invv = inv_in[...]
    a = acc_ref[0, :, :H] + acc_ref[1, :, :H]
    h = jnp.maximum(a * invv + r_ref[...], 0.0)
    p_ref[...] = _dot(h, wl_ref[...])
    rn_ref[...] = _dot(h, wr_ref[...]) + bl_ref[...]

  out_shape = [jax.ShapeDtypeStruct((N, H), _F32),
               jax.ShapeDtypeStruct((N, H), _F32)]
  if first:
    out_shape.append(jax.ShapeDtypeStruct((N, H), _F32))
    return pl.pallas_call(body, out_shape=out_shape, name="tc_combine0")(
        acc, r, wlT, wrT, bl)
  return pl.pallas_call(body, out_shape=out_shape, name="tc_combine")(
      acc, inv, r, wlT, wrT, bl)


def _tc_head(acc, inv, r, whT, bh):
  def body(acc_ref, inv_ref, r_ref, wh_ref, bh_ref, o_ref):
    a = acc_ref[0] + acc_ref[1]
    h = jnp.maximum(a * inv_ref[...] + r_ref[...], 0.0)
    o_ref[...] = _dot(h, wh_ref[...]) + bh_ref[...]

  return pl.pallas_call(
      body,
      out_shape=jax.ShapeDtypeStruct((N, OUT), _F32),
      name="tc_head",
  )(acc, inv, r, whT, bh)


def kernel(x, edge_index, Wl0, bl0, Wr0, Wl1, bl1, Wr1, Wl2, bl2, Wr2,
           Wh, bh):
  src_t = edge_index[0].reshape(NW, C, CHUNK)
  dst_t = edge_index[1].reshape(NW, C, CHUNK)

  zeros_w0 = jnp.zeros((N, W0), _F32)
  zeros_h = jnp.zeros((N, H), _F32)

  wl0T, wr0T = Wl0.T, Wr0.T
  wl1T, wr1T = Wl1.T, Wr1.T
  wl2T, wr2T = Wl2.T, Wr2.T
  whT = Wh.T
  bl0r, bl1r, bl2r, bhr = (b.reshape(1, -1) for b in (bl0, bl1, bl2, bh))

  p0, r0 = _tc_proj(x, wl0T, wr0T, bl0r)
  acc0 = _sc_aggregate(p0, src_t, dst_t, zeros_w0, W0)
  p1, r1, inv = _tc_combine(acc0, None, r0, wl1T, wr1T, bl1r)
  acc1 = _sc_aggregate(p1, src_t, dst_t, zeros_h, H)
  p2, r2 = _tc_combine(acc1, inv, r1, wl2T, wr2T, bl2r)
  acc2 = _sc_aggregate(p2, src_t, dst_t, zeros_h, H)
  return _tc_head(acc2, inv, r2, whT, bhr)
